# Initial kernel scaffold; baseline (speedup 1.0000x reference)
import jax
import jax.numpy as jnp
from jax.experimental import pallas as pl


def kernel(x, edge_index, W1, a_src1, a_dst1, b1, W2, a_src2, a_dst2, b2, Wih, Whh, bih, bhh, Wa, ba, Wf, bf):
    raise NotImplementedError("write your pallas kernel here")


# speedup vs baseline: 1.4749x; 1.4749x over previous
import functools

import jax
import jax.numpy as jnp
from jax.experimental import pallas as pl
from jax.experimental.pallas import tpu as pltpu

N = 10000
E = 320000
D = 128
H = 64
C = 2

LSTM_BLK = 8


def _prep_body(x_ref, w_ref, asrc_ref, adst_ref, h_ref, al_src_ref, al_dst_ref):
    h = jnp.dot(x_ref[...], w_ref[...], preferred_element_type=jnp.float32)
    h_ref[...] = h
    al_src_ref[...] = h @ asrc_ref[...]
    al_dst_ref[...] = h @ adst_ref[...]


def _prep(x, W, a_src, a_dst):
    n, _ = x.shape
    return pl.pallas_call(
        _prep_body,
        out_shape=(
            jax.ShapeDtypeStruct((n, H), jnp.float32),
            jax.ShapeDtypeStruct((n, 1), jnp.float32),
            jax.ShapeDtypeStruct((n, 1), jnp.float32),
        ),
    )(x, W, a_src[:, None], a_dst[:, None])


def _gat_edges(h, alpha_src, alpha_dst, src, dst, b):
    n = h.shape[0]
    e = jax.nn.leaky_relu(alpha_src[src] + alpha_dst[dst], negative_slope=0.2)
    emax = jax.ops.segment_max(e, dst, num_segments=n)
    emax = jnp.where(jnp.isfinite(emax), emax, 0.0)
    w = jnp.exp(e - emax[dst])
    denom = jax.ops.segment_sum(w, dst, num_segments=n)
    alpha = w / (denom[dst] + 1e-16)
    out = jax.ops.segment_sum(alpha[:, None] * h[src], dst, num_segments=n)
    return jax.nn.relu(out + b)


def _lstm_head_body(h_ref, wih_ref, whh_ref, b_ref, wa_ref, ba_ref, wf_ref,
                    bf_ref, out_ref, g_ref, hs_ref):
    n = h_ref.shape[0]
    g_ref[...] = (
        jnp.dot(h_ref[...], wih_ref[...], preferred_element_type=jnp.float32)
        + b_ref[...]
    )
    whh = whh_ref[...]

    def step(carry, g):
        hprev, cprev = carry
        gates = g + jnp.dot(hprev, whh, preferred_element_type=jnp.float32)
        i = jax.nn.sigmoid(gates[:, 0:H])
        f = jax.nn.sigmoid(gates[:, H:2 * H])
        gg = jnp.tanh(gates[:, 2 * H:3 * H])
        o = jax.nn.sigmoid(gates[:, 3 * H:4 * H])
        cnew = f * cprev + i * gg
        hnew = o * jnp.tanh(cnew)
        return (hnew, cnew), hnew

    def blk(k, carry):
        gblk = g_ref[pl.ds(k * LSTM_BLK, LSTM_BLK), :]
        rows = []
        for j in range(LSTM_BLK):
            carry, hnew = step(carry, gblk[j:j + 1, :])
            rows.append(hnew)
        hs_ref[pl.ds(k * LSTM_BLK, LSTM_BLK), :] = jnp.concatenate(rows, axis=0)
        return carry

    zero = jnp.zeros((1, H), jnp.float32)
    jax.lax.fori_loop(0, n // LSTM_BLK, blk, (zero, zero), unroll=False)

    hs = hs_ref[...]
    scores = jnp.dot(hs, wa_ref[...], preferred_element_type=jnp.float32) + ba_ref[0, 0]
    m = jnp.max(scores)
    wexp = jnp.exp(scores - m)
    denom = jnp.sum(wexp)
    pooled = jnp.dot(wexp.T, hs, preferred_element_type=jnp.float32) / denom
    logits = jnp.dot(pooled, wf_ref[...], preferred_element_type=jnp.float32) + bf_ref[...]
    lmax = jnp.max(logits, axis=1, keepdims=True)
    lexp = jnp.exp(logits - lmax)
    out_ref[...] = lexp / jnp.sum(lexp, axis=1, keepdims=True)


def _lstm_head(h, WihT, WhhT, bsum, Wa, ba, Wf, bf):
    n = h.shape[0]
    return pl.pallas_call(
        _lstm_head_body,
        out_shape=jax.ShapeDtypeStruct((1, C), jnp.float32),
        scratch_shapes=[
            pltpu.VMEM((n, 4 * H), jnp.float32),
            pltpu.VMEM((n, H), jnp.float32),
        ],
    )(h, WihT, WhhT, bsum, Wa, ba, Wf, bf)


def kernel(x, edge_index, W1, a_src1, a_dst1, b1, W2, a_src2, a_dst2, b2,
           Wih, Whh, bih, bhh, Wa, ba, Wf, bf):
    src = edge_index[0]
    dst = edge_index[1]

    h1, asrc1, adst1 = _prep(x, W1, a_src1, a_dst1)
    h1o = _gat_edges(h1, asrc1[:, 0], adst1[:, 0], src, dst, b1)

    h2, asrc2, adst2 = _prep(h1o, W2, a_src2, a_dst2)
    h2o = _gat_edges(h2, asrc2[:, 0], adst2[:, 0], src, dst, b2)

    bsum = (bih + bhh)[None, :]
    out = _lstm_head(h2o, Wih.T, Whh.T, bsum, Wa, ba[:, None], Wf, bf[None, :])
    return out
